# baseline (speedup 1.0000x reference)
import functools

import jax
import jax.numpy as jnp
from jax import lax
from jax.experimental import pallas as pl
from jax.experimental.pallas import tpu as pltpu
from jax.experimental.pallas import tpu_sc as plsc

EMB = 32
CW = 512
NC, NS = 2, 16
NW = NC * NS
L = 16
ET, EI = EMB // 8, 8
BI = 128


def _emb_kernel(n_seq, x_hbm, tab_hbm, out_hbm, idx_v, rows_v, trows_v,
                gs0, gs1, os0, os1):
    wid = lax.axis_index("s") * NC + lax.axis_index("c")
    boff = wid * CW
    bt0 = wid * (CW // BI)
    gsems = (gs0, gs1)
    osems = (os0, os1)

    def fire(buf, s):
        pltpu.sync_copy(x_hbm.at[s, pl.ds(boff, CW)], idx_v.at[buf])
        for j in range(CW // L):
            idx_v[buf, pl.ds(j * L, L)] = idx_v[buf, pl.ds(j * L, L)] * 4
        return pltpu.async_copy(tab_hbm.at[idx_v.at[buf]], rows_v.at[buf],
                                gsems[buf])

    lane = jnp.arange(L, dtype=jnp.int32)

    def transpose(buf):
        bufv = jnp.full((L,), buf, dtype=jnp.int32)

        def tbody(b0, _):
            bvec = lane + b0 * L
            bt = b0 // (BI // L)
            bi = (b0 % (BI // L)) * L
            for e in range(EMB):
                v = plsc.load_gather(
                    rows_v, [bufv, bvec, jnp.full((L,), e, dtype=jnp.int32)])
                trows_v[buf, e // EI, bt, e % EI, pl.ds(bi, L)] = v
            return _

        lax.fori_loop(0, CW // L, tbody, None)

    def write(buf, s):
        return pltpu.async_copy(
            trows_v.at[buf],
            out_hbm.at[s, :, pl.ds(bt0, CW // BI), :, :], osems[buf])

    def wait_writes():
        pltpu.make_async_copy(
            trows_v.at[0], out_hbm.at[0, :, pl.ds(bt0, CW // BI), :, :],
            os0).wait()
        pltpu.make_async_copy(
            trows_v.at[1], out_hbm.at[0, :, pl.ds(bt0, CW // BI), :, :],
            os1).wait()

    def body(i, _):
        s0 = 2 * i
        h0 = fire(0, s0)
        h1 = fire(1, s0 + 1)
        pl.when(i > 0)(wait_writes)
        h0.wait()
        transpose(0)
        write(0, s0)
        h1.wait()
        transpose(1)
        write(1, s0 + 1)
        return _

    lax.fori_loop(0, n_seq // 2, body, None)
    wait_writes()


@jax.jit
def _emb(xt, table4):
    n_seq, n_batch = xt.shape
    mesh = plsc.VectorSubcoreMesh(core_axis_name="c", subcore_axis_name="s",
                                  num_cores=NC, num_subcores=NS)
    k = pl.kernel(
        functools.partial(_emb_kernel, n_seq),
        out_type=jax.ShapeDtypeStruct((n_seq, ET, n_batch // BI, EI, BI),
                                      jnp.float32),
        mesh=mesh,
        scratch_types=[
            pltpu.VMEM((2, CW), jnp.int32),
            pltpu.VMEM((2, CW, EMB), jnp.float32),
            pltpu.VMEM((2, ET, CW // BI, EI, BI), jnp.float32),
            pltpu.SemaphoreType.DMA,
            pltpu.SemaphoreType.DMA,
            pltpu.SemaphoreType.DMA,
            pltpu.SemaphoreType.DMA,
        ],
        compiler_params=pltpu.CompilerParams(use_tc_tiling_on_sc=False,
                                             needs_layout_passes=False),
    )
    return k(xt, table4)


def kernel(x, table):
    n_batch, n_seq = x.shape
    table4 = jnp.pad(table, ((0, 0), (0, 128 - EMB))).reshape(-1, EMB)
    out5 = _emb(x.T, table4)
    return out5.transpose(2, 4, 0, 1, 3).reshape(n_batch, n_seq, EMB)
